# baseline (speedup 1.0000x reference)
import functools

import jax
import jax.numpy as jnp
from jax.experimental import pallas as pl
from jax.experimental.pallas import tpu as pltpu

_BN_EPS = 1e-5
_PAD = 128


def _fused_block_kernel(x_ref, w_ref, s_ref, b_ref, pw_ref, b2_ref, o_ref,
                        xpad_ref, *, hw, w_img, kh, kw):
    c = x_ref.shape[1]
    xpad_ref[:, :_PAD] = jnp.zeros((c, _PAD), jnp.float32)
    xpad_ref[:, _PAD + hw:] = jnp.zeros((c, _PAD), jnp.float32)
    xpad_ref[:, _PAD:_PAD + hw] = x_ref[0].astype(jnp.float32)

    ph, pw_pad = kh // 2, kw // 2
    col = jax.lax.broadcasted_iota(jnp.int32, (c, hw), 1) % w_img

    acc = None
    for j in range(kw):
        dj = j - pw_pad
        g = None
        for i in range(kh):
            d = w_img * (i - ph) + dj
            xs = xpad_ref[:, _PAD + d:_PAD + d + hw]
            term = xs * w_ref[:, kw * i + j:kw * i + j + 1]
            g = term if g is None else g + term
        if dj < 0:
            g = jnp.where(col >= -dj, g, 0.0)
        elif dj > 0:
            g = jnp.where(col < w_img - dj, g, 0.0)
        acc = g if acc is None else acc + g

    mid = jnp.maximum(acc * s_ref[...] + b_ref[...], 0.0).astype(jnp.bfloat16)
    y = jnp.dot(pw_ref[...], mid, preferred_element_type=jnp.float32)
    o_ref[0] = jnp.maximum(y + b2_ref[...], 0.0).astype(o_ref.dtype)


def kernel(x, dw_w, pw_w, bn1_gamma, bn1_beta, bn1_mean, bn1_var,
           bn2_gamma, bn2_beta, bn2_mean, bn2_var):
    n, c_in, h, w = x.shape
    kh, kw = int(dw_w.shape[2]), int(dw_w.shape[3])
    c_out = pw_w.shape[0]
    hw = h * w

    s1 = bn1_gamma / jnp.sqrt(bn1_var + _BN_EPS)
    b1 = bn1_beta - bn1_mean * s1
    s2 = bn2_gamma / jnp.sqrt(bn2_var + _BN_EPS)
    b2 = bn2_beta - bn2_mean * s2
    pw_folded = (pw_w.reshape(c_out, c_in) * s2[:, None]).astype(jnp.bfloat16)

    x_flat = x.reshape(n, c_in, hw)
    w_taps = dw_w.reshape(c_in, kh * kw)

    body = functools.partial(_fused_block_kernel, hw=hw, w_img=w, kh=kh, kw=kw)
    out_flat = pl.pallas_call(
        body,
        out_shape=jax.ShapeDtypeStruct((n, c_out, hw), x.dtype),
        grid=(n,),
        in_specs=[
            pl.BlockSpec((1, c_in, hw), lambda b: (b, 0, 0)),
            pl.BlockSpec((c_in, kh * kw), lambda b: (0, 0)),
            pl.BlockSpec((c_in, 1), lambda b: (0, 0)),
            pl.BlockSpec((c_in, 1), lambda b: (0, 0)),
            pl.BlockSpec((c_out, c_in), lambda b: (0, 0)),
            pl.BlockSpec((c_out, 1), lambda b: (0, 0)),
        ],
        out_specs=pl.BlockSpec((1, c_out, hw), lambda b: (b, 0, 0)),
        scratch_shapes=[pltpu.VMEM((c_in, hw + 2 * _PAD), jnp.float32)],
        compiler_params=pltpu.CompilerParams(dimension_semantics=("parallel",)),
    )(x_flat, w_taps, s1.reshape(c_in, 1), b1.reshape(c_in, 1),
      pw_folded, b2.reshape(c_out, 1))
    return out_flat.reshape(n, c_out, h, w)
